# baseline (speedup 1.0000x reference)
import functools

import jax
import jax.numpy as jnp
from jax import lax
from jax.experimental import pallas as pl
from jax.experimental.pallas import tpu as pltpu
from jax.experimental.pallas import tpu_sc as plsc

N = 10000
E = 320000
D = 128
H = 128
NE = 8

NC = 2
NS = 16
NW = NC * NS
CHUNK = 50
CPT = E // (NW * CHUNK)
GK = 8
NG = CPT // GK
NSLOT = 4
NPAD = 10240
ROWS_PER_TILE = NPAD // NS
NBLK = 1000
GRID = N // NBLK


def _segsum_body(x_hbm, src_hbm, dst_hbm, zer_hbm, out_hbm,
                 src_v, dst_v, rows_v, acc_sh,
                 semi0, semi1, semr0, semr1, semr2, semr3,
                 sems0, sems1, sems2, sems3):
    c = lax.axis_index("c")
    s = lax.axis_index("s")
    tile = c * NS + s
    semi = (semi0, semi1)
    semr = (semr0, semr1, semr2, semr3)
    sems = (sems0, sems1, sems2, sems3)

    pltpu.sync_copy(zer_hbm, acc_sh.at[pl.ds(s * ROWS_PER_TILE, ROWS_PER_TILE)])

    pltpu.async_copy(src_hbm.at[tile, 0], src_v.at[0], semi0)
    pltpu.async_copy(dst_hbm.at[tile, 0], dst_v.at[0], semi0)
    plsc.subcore_barrier()
    pltpu.make_async_copy(src_hbm.at[tile, 0], src_v.at[0], semi0).wait()
    pltpu.make_async_copy(dst_hbm.at[tile, 0], dst_v.at[0], semi0).wait()
    pltpu.async_copy(x_hbm.at[src_v.at[0, 0]], rows_v.at[0], semr0)

    def _scatter_wait(rb, gb, k):
        pltpu.make_async_copy(rows_v.at[rb], acc_sh.at[dst_v.at[gb, k]],
                              sems[rb]).wait()

    def _group(g, gb, first):
        for k in range(GK):
            rb = k % NSLOT
            nb = (k + 1) % NSLOT
            if not (first and k < NSLOT - 1):
                if k >= NSLOT - 1:
                    _scatter_wait(nb, gb, k + 1 - NSLOT)
                else:
                    _scatter_wait(nb, 1 - gb, GK + k + 1 - NSLOT)
            if k < GK - 1:
                pltpu.async_copy(x_hbm.at[src_v.at[gb, k + 1]],
                                 rows_v.at[nb], semr[nb])
            else:
                @pl.when(g + 1 < NG)
                def _():
                    pltpu.make_async_copy(src_hbm.at[tile, g + 1],
                                          src_v.at[1 - gb], semi[1 - gb]).wait()
                    pltpu.make_async_copy(dst_hbm.at[tile, g + 1],
                                          dst_v.at[1 - gb], semi[1 - gb]).wait()
                    pltpu.async_copy(x_hbm.at[src_v.at[1 - gb, 0]],
                                     rows_v.at[nb], semr[nb])
            pltpu.make_async_copy(x_hbm.at[src_v.at[gb, k]], rows_v.at[rb],
                                  semr[rb]).wait()
            pltpu.async_copy(rows_v.at[rb], acc_sh.at[dst_v.at[gb, k]],
                             sems[rb], add=True)
            if k == NSLOT - 2:
                @pl.when(g + 1 < NG)
                def _():
                    pltpu.async_copy(src_hbm.at[tile, g + 1], src_v.at[1 - gb],
                                     semi[1 - gb])
                    pltpu.async_copy(dst_hbm.at[tile, g + 1], dst_v.at[1 - gb],
                                     semi[1 - gb])

    _group(0, 0, True)

    @pl.loop(1, NG, step=2)
    def _(g):
        _group(g, 1, False)

        @pl.when(g + 1 < NG)
        def _():
            _group(g + 1, 0, False)

    for k in range(GK - NSLOT + 1, GK):
        _scatter_wait(k % NSLOT, (NG - 1) % 2, k)
    plsc.subcore_barrier()
    pltpu.sync_copy(acc_sh.at[pl.ds(s * ROWS_PER_TILE, ROWS_PER_TILE)],
                    out_hbm.at[c, pl.ds(s * ROWS_PER_TILE, ROWS_PER_TILE)])


@functools.cache
def _make_segsum():
    return pl.kernel(
        _segsum_body,
        out_type=jax.ShapeDtypeStruct((NC, NPAD, D), jnp.float32),
        mesh=plsc.VectorSubcoreMesh(core_axis_name="c", subcore_axis_name="s",
                                    num_cores=NC, num_subcores=NS),
        scratch_types=[
            pltpu.VMEM((2, GK, CHUNK), jnp.int32),
            pltpu.VMEM((2, GK, CHUNK), jnp.int32),
            pltpu.VMEM((NSLOT, CHUNK, D), jnp.float32),
            pltpu.VMEM_SHARED((NPAD, D), jnp.float32),
        ] + [pltpu.SemaphoreType.DMA] * 10,
    )


def _mlp1_body(eps_ref, x_ref, a0_ref, a1_ref, wa_ref, ba_ref, wb_ref, bb_ref, o_ref):
    h = (1.0 + eps_ref[0, 0]) * x_ref[...] + a0_ref[0] + a1_ref[0]
    h = jnp.maximum(jnp.dot(h, wa_ref[...], preferred_element_type=jnp.float32)
                    + ba_ref[...], 0.0)
    h = jnp.maximum(jnp.dot(h, wb_ref[...], preferred_element_type=jnp.float32)
                    + bb_ref[...], 0.0)
    o_ref[...] = h


def _mlp2_body(eps_ref, x_ref, a0_ref, a1_ref, wa_ref, ba_ref, wb_ref, bb_ref,
               hw1_ref, hb1_ref, hw2_ref, hb2_ref, bnd_ref, o_ref, csum_ref):
    i = pl.program_id(0)
    h = (1.0 + eps_ref[0, 0]) * x_ref[...] + a0_ref[0] + a1_ref[0]
    h = jnp.maximum(jnp.dot(h, wa_ref[...], preferred_element_type=jnp.float32)
                    + ba_ref[...], 0.0)
    h = jnp.maximum(jnp.dot(h, wb_ref[...], preferred_element_type=jnp.float32)
                    + bb_ref[...], 0.0)
    cs = jnp.sum(h, axis=0, keepdims=True)

    @pl.when(i == 0)
    def _():
        csum_ref[...] = cs

    @pl.when(i > 0)
    def _():
        csum_ref[...] = csum_ref[...] + cs

    @pl.when(i == GRID - 1)
    def _():
        hp = csum_ref[...] * (1.0 / N)
        rows = []
        for e in range(NE):
            z = jnp.maximum(
                jnp.dot(hp, hw1_ref[e], preferred_element_type=jnp.float32)
                + hb1_ref[e:e + 1, :], 0.0)
            r = jax.nn.sigmoid(
                jnp.dot(z, hw2_ref[e], preferred_element_type=jnp.float32)
                + hb2_ref[e:e + 1, :])
            rows.append(jnp.minimum(r, bnd_ref[e:e + 1, :]))
        o_ref[...] = jnp.concatenate(rows, axis=0)


def _row_blocks():
    return [
        pl.BlockSpec((1, 1), lambda i: (0, 0), memory_space=pltpu.SMEM),
        pl.BlockSpec((NBLK, D), lambda i: (i, 0)),
        pl.BlockSpec((1, NBLK, D), lambda i: (0, i, 0)),
        pl.BlockSpec((1, NBLK, D), lambda i: (1, i, 0)),
        pl.BlockSpec((D, H), lambda i: (0, 0)),
        pl.BlockSpec((1, H), lambda i: (0, 0)),
        pl.BlockSpec((H, H), lambda i: (0, 0)),
        pl.BlockSpec((1, H), lambda i: (0, 0)),
    ]


_mlp1 = pl.pallas_call(
    _mlp1_body,
    grid=(GRID,),
    in_specs=_row_blocks(),
    out_specs=pl.BlockSpec((NBLK, H), lambda i: (i, 0)),
    out_shape=jax.ShapeDtypeStruct((N, H), jnp.float32),
)

_mlp2 = pl.pallas_call(
    _mlp2_body,
    grid=(GRID,),
    in_specs=_row_blocks() + [
        pl.BlockSpec((NE, H, H), lambda i: (0, 0, 0)),
        pl.BlockSpec((NE, H), lambda i: (0, 0)),
        pl.BlockSpec((NE, H, H), lambda i: (0, 0, 0)),
        pl.BlockSpec((NE, H), lambda i: (0, 0)),
        pl.BlockSpec((NE, H), lambda i: (0, 0)),
    ],
    out_specs=pl.BlockSpec((NE, H), lambda i: (0, 0)),
    out_shape=jax.ShapeDtypeStruct((NE, H), jnp.float32),
    scratch_shapes=[pltpu.VMEM((1, H), jnp.float32)],
)


def kernel(x, edge_index, eps0, eps1, W0a, b0a, W0b, b0b, W1a, b1a, W1b, b1b,
           HW1, Hb1, HW2, Hb2):
    src = edge_index[0].astype(jnp.int32).reshape(NW, NG, GK, CHUNK)
    dst = edge_index[1].astype(jnp.int32).reshape(NW, NG, GK, CHUNK)
    zer = jnp.zeros((ROWS_PER_TILE, D), jnp.float32)

    eps0r = jnp.reshape(eps0, (1, 1))
    eps1r = jnp.reshape(eps1, (1, 1))
    b0ar = b0a.reshape(1, H)
    b0br = b0b.reshape(1, H)
    b1ar = b1a.reshape(1, H)
    b1br = b1b.reshape(1, H)
    HW2p = jnp.pad(HW2, ((0, 0), (0, 0), (0, H - 4)))
    Hb2p = jnp.pad(Hb2, ((0, 0), (0, H - 4)))
    bounds = jnp.array([0.3, 0.3, 0.3, 0.1], jnp.float32)
    bndp = jnp.pad(jnp.broadcast_to(bounds, (NE, 4)), ((0, 0), (0, H - 4)),
                   constant_values=1.0)

    segsum = _make_segsum()
    agg1 = segsum(x, src, dst, zer)
    h1 = _mlp1(eps0r, x, agg1, agg1, W0a, b0ar, W0b, b0br)
    agg2 = segsum(h1, src, dst, zer)
    out = _mlp2(eps1r, h1, agg2, agg2, W1a, b1ar, W1b, b1br,
                HW1, Hb1, HW2p, Hb2p, bndp)
    return out[:, :4]


# speedup vs baseline: 1.1552x; 1.1552x over previous
import functools

import jax
import jax.numpy as jnp
from jax import lax
from jax.experimental import pallas as pl
from jax.experimental.pallas import tpu as pltpu
from jax.experimental.pallas import tpu_sc as plsc

N = 10000
E = 320000
D = 128
H = 128
NE = 8

NC = 2
NS = 16
NW = NC * NS
CHUNK = 100
CPT = E // (NW * CHUNK)
GK = 10
NG = CPT // GK
NSLOT = 2
NPAD = 10240
ROWS_PER_TILE = NPAD // NS
NBLK = 1000
GRID = N // NBLK


def _segsum_body(x_hbm, src_hbm, dst_hbm, zer_hbm, out_hbm,
                 src_v, dst_v, rows_v, acc_sh,
                 semi0, semi1, semr0, semr1, semr2, semr3,
                 sems0, sems1, sems2, sems3):
    c = lax.axis_index("c")
    s = lax.axis_index("s")
    tile = c * NS + s
    semi = (semi0, semi1)
    semr = (semr0, semr1, semr2, semr3)[:NSLOT]
    sems = (sems0, sems1, sems2, sems3)[:NSLOT]

    pltpu.sync_copy(zer_hbm, acc_sh.at[pl.ds(s * ROWS_PER_TILE, ROWS_PER_TILE)])

    pltpu.async_copy(src_hbm.at[tile, 0], src_v.at[0], semi0)
    pltpu.async_copy(dst_hbm.at[tile, 0], dst_v.at[0], semi0)
    plsc.subcore_barrier()
    pltpu.make_async_copy(src_hbm.at[tile, 0], src_v.at[0], semi0).wait()
    pltpu.make_async_copy(dst_hbm.at[tile, 0], dst_v.at[0], semi0).wait()
    pltpu.async_copy(x_hbm.at[src_v.at[0, 0]], rows_v.at[0], semr0)

    def _scatter_wait(rb, gb, k):
        pltpu.make_async_copy(rows_v.at[rb], acc_sh.at[dst_v.at[gb, k]],
                              sems[rb]).wait()

    def _group(g, gb, first):
        for k in range(GK):
            rb = k % NSLOT
            nb = (k + 1) % NSLOT
            if not (first and k < NSLOT - 1):
                if k >= NSLOT - 1:
                    _scatter_wait(nb, gb, k + 1 - NSLOT)
                else:
                    _scatter_wait(nb, 1 - gb, GK + k + 1 - NSLOT)
            if k < GK - 1:
                pltpu.async_copy(x_hbm.at[src_v.at[gb, k + 1]],
                                 rows_v.at[nb], semr[nb])
            else:
                @pl.when(g + 1 < NG)
                def _():
                    pltpu.make_async_copy(src_hbm.at[tile, g + 1],
                                          src_v.at[1 - gb], semi[1 - gb]).wait()
                    pltpu.make_async_copy(dst_hbm.at[tile, g + 1],
                                          dst_v.at[1 - gb], semi[1 - gb]).wait()
                    pltpu.async_copy(x_hbm.at[src_v.at[1 - gb, 0]],
                                     rows_v.at[nb], semr[nb])
            pltpu.make_async_copy(x_hbm.at[src_v.at[gb, k]], rows_v.at[rb],
                                  semr[rb]).wait()
            pltpu.async_copy(rows_v.at[rb], acc_sh.at[dst_v.at[gb, k]],
                             sems[rb], add=True)
            if k == NSLOT - 2:
                @pl.when(g + 1 < NG)
                def _():
                    pltpu.async_copy(src_hbm.at[tile, g + 1], src_v.at[1 - gb],
                                     semi[1 - gb])
                    pltpu.async_copy(dst_hbm.at[tile, g + 1], dst_v.at[1 - gb],
                                     semi[1 - gb])

    _group(0, 0, True)

    @pl.loop(1, NG, step=2)
    def _(g):
        _group(g, 1, False)

        @pl.when(g + 1 < NG)
        def _():
            _group(g + 1, 0, False)

    for k in range(GK - NSLOT + 1, GK):
        _scatter_wait(k % NSLOT, (NG - 1) % 2, k)
    plsc.subcore_barrier()
    pltpu.sync_copy(acc_sh.at[pl.ds(s * ROWS_PER_TILE, ROWS_PER_TILE)],
                    out_hbm.at[c, pl.ds(s * ROWS_PER_TILE, ROWS_PER_TILE)])


@functools.cache
def _make_segsum():
    return pl.kernel(
        _segsum_body,
        out_type=jax.ShapeDtypeStruct((NC, NPAD, D), jnp.float32),
        mesh=plsc.VectorSubcoreMesh(core_axis_name="c", subcore_axis_name="s",
                                    num_cores=NC, num_subcores=NS),
        scratch_types=[
            pltpu.VMEM((2, GK, CHUNK), jnp.int32),
            pltpu.VMEM((2, GK, CHUNK), jnp.int32),
            pltpu.VMEM((NSLOT, CHUNK, D), jnp.float32),
            pltpu.VMEM_SHARED((NPAD, D), jnp.float32),
        ] + [pltpu.SemaphoreType.DMA] * 10,
    )


def _mlp1_body(eps_ref, x_ref, a0_ref, a1_ref, wa_ref, ba_ref, wb_ref, bb_ref, o_ref):
    h = (1.0 + eps_ref[0, 0]) * x_ref[...] + a0_ref[0] + a1_ref[0]
    h = jnp.maximum(jnp.dot(h, wa_ref[...], preferred_element_type=jnp.float32)
                    + ba_ref[...], 0.0)
    h = jnp.maximum(jnp.dot(h, wb_ref[...], preferred_element_type=jnp.float32)
                    + bb_ref[...], 0.0)
    o_ref[...] = h


def _mlp2_body(eps_ref, x_ref, a0_ref, a1_ref, wa_ref, ba_ref, wb_ref, bb_ref,
               hw1_ref, hb1_ref, hw2_ref, hb2_ref, bnd_ref, o_ref, csum_ref):
    i = pl.program_id(0)
    h = (1.0 + eps_ref[0, 0]) * x_ref[...] + a0_ref[0] + a1_ref[0]
    h = jnp.maximum(jnp.dot(h, wa_ref[...], preferred_element_type=jnp.float32)
                    + ba_ref[...], 0.0)
    h = jnp.maximum(jnp.dot(h, wb_ref[...], preferred_element_type=jnp.float32)
                    + bb_ref[...], 0.0)
    cs = jnp.sum(h, axis=0, keepdims=True)

    @pl.when(i == 0)
    def _():
        csum_ref[...] = cs

    @pl.when(i > 0)
    def _():
        csum_ref[...] = csum_ref[...] + cs

    @pl.when(i == GRID - 1)
    def _():
        hp = csum_ref[...] * (1.0 / N)
        rows = []
        for e in range(NE):
            z = jnp.maximum(
                jnp.dot(hp, hw1_ref[e], preferred_element_type=jnp.float32)
                + hb1_ref[e:e + 1, :], 0.0)
            r = jax.nn.sigmoid(
                jnp.dot(z, hw2_ref[e], preferred_element_type=jnp.float32)
                + hb2_ref[e:e + 1, :])
            rows.append(jnp.minimum(r, bnd_ref[e:e + 1, :]))
        o_ref[...] = jnp.concatenate(rows, axis=0)


def _row_blocks():
    return [
        pl.BlockSpec((1, 1), lambda i: (0, 0), memory_space=pltpu.SMEM),
        pl.BlockSpec((NBLK, D), lambda i: (i, 0)),
        pl.BlockSpec((1, NBLK, D), lambda i: (0, i, 0)),
        pl.BlockSpec((1, NBLK, D), lambda i: (1, i, 0)),
        pl.BlockSpec((D, H), lambda i: (0, 0)),
        pl.BlockSpec((1, H), lambda i: (0, 0)),
        pl.BlockSpec((H, H), lambda i: (0, 0)),
        pl.BlockSpec((1, H), lambda i: (0, 0)),
    ]


_mlp1 = pl.pallas_call(
    _mlp1_body,
    grid=(GRID,),
    in_specs=_row_blocks(),
    out_specs=pl.BlockSpec((NBLK, H), lambda i: (i, 0)),
    out_shape=jax.ShapeDtypeStruct((N, H), jnp.float32),
)

_mlp2 = pl.pallas_call(
    _mlp2_body,
    grid=(GRID,),
    in_specs=_row_blocks() + [
        pl.BlockSpec((NE, H, H), lambda i: (0, 0, 0)),
        pl.BlockSpec((NE, H), lambda i: (0, 0)),
        pl.BlockSpec((NE, H, H), lambda i: (0, 0, 0)),
        pl.BlockSpec((NE, H), lambda i: (0, 0)),
        pl.BlockSpec((NE, H), lambda i: (0, 0)),
    ],
    out_specs=pl.BlockSpec((NE, H), lambda i: (0, 0)),
    out_shape=jax.ShapeDtypeStruct((NE, H), jnp.float32),
    scratch_shapes=[pltpu.VMEM((1, H), jnp.float32)],
)


def kernel(x, edge_index, eps0, eps1, W0a, b0a, W0b, b0b, W1a, b1a, W1b, b1b,
           HW1, Hb1, HW2, Hb2):
    src = edge_index[0].astype(jnp.int32).reshape(NW, NG, GK, CHUNK)
    dst = edge_index[1].astype(jnp.int32).reshape(NW, NG, GK, CHUNK)
    zer = jnp.zeros((ROWS_PER_TILE, D), jnp.float32)

    eps0r = jnp.reshape(eps0, (1, 1))
    eps1r = jnp.reshape(eps1, (1, 1))
    b0ar = b0a.reshape(1, H)
    b0br = b0b.reshape(1, H)
    b1ar = b1a.reshape(1, H)
    b1br = b1b.reshape(1, H)
    HW2p = jnp.pad(HW2, ((0, 0), (0, 0), (0, H - 4)))
    Hb2p = jnp.pad(Hb2, ((0, 0), (0, H - 4)))
    bounds = jnp.array([0.3, 0.3, 0.3, 0.1], jnp.float32)
    bndp = jnp.pad(jnp.broadcast_to(bounds, (NE, 4)), ((0, 0), (0, H - 4)),
                   constant_values=1.0)

    segsum = _make_segsum()
    agg1 = segsum(x, src, dst, zer)
    h1 = _mlp1(eps0r, x, agg1, agg1, W0a, b0ar, W0b, b0br)
    agg2 = segsum(h1, src, dst, zer)
    out = _mlp2(eps1r, h1, agg2, agg2, W1a, b1ar, W1b, b1br,
                HW1, Hb1, HW2p, Hb2p, bndp)
    return out[:, :4]


# speedup vs baseline: 1.1755x; 1.0175x over previous
import functools

import jax
import jax.numpy as jnp
from jax import lax
from jax.experimental import pallas as pl
from jax.experimental.pallas import tpu as pltpu
from jax.experimental.pallas import tpu_sc as plsc

N = 10000
E = 320000
D = 128
H = 128
NE = 8

NC = 2
NS = 16
NW = NC * NS
CHUNK = 125
CPT = E // (NW * CHUNK)
GK = 8
NG = CPT // GK
NSLOT = 2
NPAD = 10240
ROWS_PER_TILE = NPAD // NS
NBLK = 1000
GRID = N // NBLK


def _segsum_body(x_hbm, src_hbm, dst_hbm, zer_hbm, out_hbm,
                 src_v, dst_v, rows_v, acc_sh,
                 semi0, semi1, semr0, semr1, semr2, semr3,
                 sems0, sems1, sems2, sems3):
    c = lax.axis_index("c")
    s = lax.axis_index("s")
    tile = c * NS + s
    semi = (semi0, semi1)
    semr = (semr0, semr1, semr2, semr3)[:NSLOT]
    sems = (sems0, sems1, sems2, sems3)[:NSLOT]

    pltpu.sync_copy(zer_hbm, acc_sh.at[pl.ds(s * ROWS_PER_TILE, ROWS_PER_TILE)])

    pltpu.async_copy(src_hbm.at[tile, 0], src_v.at[0], semi0)
    pltpu.async_copy(dst_hbm.at[tile, 0], dst_v.at[0], semi0)
    plsc.subcore_barrier()
    pltpu.make_async_copy(src_hbm.at[tile, 0], src_v.at[0], semi0).wait()
    pltpu.make_async_copy(dst_hbm.at[tile, 0], dst_v.at[0], semi0).wait()
    pltpu.async_copy(x_hbm.at[src_v.at[0, 0]], rows_v.at[0], semr0)

    def _scatter_wait(rb, gb, k):
        pltpu.make_async_copy(rows_v.at[rb], acc_sh.at[dst_v.at[gb, k]],
                              sems[rb]).wait()

    def _group(g, gb, first):
        for k in range(GK):
            rb = k % NSLOT
            nb = (k + 1) % NSLOT
            if not (first and k < NSLOT - 1):
                if k >= NSLOT - 1:
                    _scatter_wait(nb, gb, k + 1 - NSLOT)
                else:
                    _scatter_wait(nb, 1 - gb, GK + k + 1 - NSLOT)
            if k < GK - 1:
                pltpu.async_copy(x_hbm.at[src_v.at[gb, k + 1]],
                                 rows_v.at[nb], semr[nb])
            else:
                @pl.when(g + 1 < NG)
                def _():
                    pltpu.make_async_copy(src_hbm.at[tile, g + 1],
                                          src_v.at[1 - gb], semi[1 - gb]).wait()
                    pltpu.make_async_copy(dst_hbm.at[tile, g + 1],
                                          dst_v.at[1 - gb], semi[1 - gb]).wait()
                    pltpu.async_copy(x_hbm.at[src_v.at[1 - gb, 0]],
                                     rows_v.at[nb], semr[nb])
            pltpu.make_async_copy(x_hbm.at[src_v.at[gb, k]], rows_v.at[rb],
                                  semr[rb]).wait()
            pltpu.async_copy(rows_v.at[rb], acc_sh.at[dst_v.at[gb, k]],
                             sems[rb], add=True)
            if k == NSLOT - 2:
                @pl.when(g + 1 < NG)
                def _():
                    pltpu.async_copy(src_hbm.at[tile, g + 1], src_v.at[1 - gb],
                                     semi[1 - gb])
                    pltpu.async_copy(dst_hbm.at[tile, g + 1], dst_v.at[1 - gb],
                                     semi[1 - gb])

    _group(0, 0, True)

    @pl.loop(1, NG, step=2)
    def _(g):
        _group(g, 1, False)

        @pl.when(g + 1 < NG)
        def _():
            _group(g + 1, 0, False)

    for k in range(GK - NSLOT + 1, GK):
        _scatter_wait(k % NSLOT, (NG - 1) % 2, k)
    plsc.subcore_barrier()
    pltpu.sync_copy(acc_sh.at[pl.ds(s * ROWS_PER_TILE, ROWS_PER_TILE)],
                    out_hbm.at[c, pl.ds(s * ROWS_PER_TILE, ROWS_PER_TILE)])


@functools.cache
def _make_segsum():
    return pl.kernel(
        _segsum_body,
        out_type=jax.ShapeDtypeStruct((NC, NPAD, D), jnp.float32),
        mesh=plsc.VectorSubcoreMesh(core_axis_name="c", subcore_axis_name="s",
                                    num_cores=NC, num_subcores=NS),
        scratch_types=[
            pltpu.VMEM((2, GK, CHUNK), jnp.int32),
            pltpu.VMEM((2, GK, CHUNK), jnp.int32),
            pltpu.VMEM((NSLOT, CHUNK, D), jnp.float32),
            pltpu.VMEM_SHARED((NPAD, D), jnp.float32),
        ] + [pltpu.SemaphoreType.DMA] * 10,
    )


def _mlp1_body(eps_ref, x_ref, a0_ref, a1_ref, wa_ref, ba_ref, wb_ref, bb_ref, o_ref):
    h = (1.0 + eps_ref[0, 0]) * x_ref[...] + a0_ref[0] + a1_ref[0]
    h = jnp.maximum(jnp.dot(h, wa_ref[...], preferred_element_type=jnp.float32)
                    + ba_ref[...], 0.0)
    h = jnp.maximum(jnp.dot(h, wb_ref[...], preferred_element_type=jnp.float32)
                    + bb_ref[...], 0.0)
    o_ref[...] = h


def _mlp2_body(eps_ref, x_ref, a0_ref, a1_ref, wa_ref, ba_ref, wb_ref, bb_ref,
               hw1_ref, hb1_ref, hw2_ref, hb2_ref, bnd_ref, o_ref, csum_ref):
    i = pl.program_id(0)
    h = (1.0 + eps_ref[0, 0]) * x_ref[...] + a0_ref[0] + a1_ref[0]
    h = jnp.maximum(jnp.dot(h, wa_ref[...], preferred_element_type=jnp.float32)
                    + ba_ref[...], 0.0)
    h = jnp.maximum(jnp.dot(h, wb_ref[...], preferred_element_type=jnp.float32)
                    + bb_ref[...], 0.0)
    cs = jnp.sum(h, axis=0, keepdims=True)

    @pl.when(i == 0)
    def _():
        csum_ref[...] = cs

    @pl.when(i > 0)
    def _():
        csum_ref[...] = csum_ref[...] + cs

    @pl.when(i == GRID - 1)
    def _():
        hp = csum_ref[...] * (1.0 / N)
        rows = []
        for e in range(NE):
            z = jnp.maximum(
                jnp.dot(hp, hw1_ref[e], preferred_element_type=jnp.float32)
                + hb1_ref[e:e + 1, :], 0.0)
            r = jax.nn.sigmoid(
                jnp.dot(z, hw2_ref[e], preferred_element_type=jnp.float32)
                + hb2_ref[e:e + 1, :])
            rows.append(jnp.minimum(r, bnd_ref[e:e + 1, :]))
        o_ref[...] = jnp.concatenate(rows, axis=0)


def _row_blocks():
    return [
        pl.BlockSpec((1, 1), lambda i: (0, 0), memory_space=pltpu.SMEM),
        pl.BlockSpec((NBLK, D), lambda i: (i, 0)),
        pl.BlockSpec((1, NBLK, D), lambda i: (0, i, 0)),
        pl.BlockSpec((1, NBLK, D), lambda i: (1, i, 0)),
        pl.BlockSpec((D, H), lambda i: (0, 0)),
        pl.BlockSpec((1, H), lambda i: (0, 0)),
        pl.BlockSpec((H, H), lambda i: (0, 0)),
        pl.BlockSpec((1, H), lambda i: (0, 0)),
    ]


_mlp1 = pl.pallas_call(
    _mlp1_body,
    grid=(GRID,),
    in_specs=_row_blocks(),
    out_specs=pl.BlockSpec((NBLK, H), lambda i: (i, 0)),
    out_shape=jax.ShapeDtypeStruct((N, H), jnp.float32),
)

_mlp2 = pl.pallas_call(
    _mlp2_body,
    grid=(GRID,),
    in_specs=_row_blocks() + [
        pl.BlockSpec((NE, H, H), lambda i: (0, 0, 0)),
        pl.BlockSpec((NE, H), lambda i: (0, 0)),
        pl.BlockSpec((NE, H, H), lambda i: (0, 0, 0)),
        pl.BlockSpec((NE, H), lambda i: (0, 0)),
        pl.BlockSpec((NE, H), lambda i: (0, 0)),
    ],
    out_specs=pl.BlockSpec((NE, H), lambda i: (0, 0)),
    out_shape=jax.ShapeDtypeStruct((NE, H), jnp.float32),
    scratch_shapes=[pltpu.VMEM((1, H), jnp.float32)],
)


def kernel(x, edge_index, eps0, eps1, W0a, b0a, W0b, b0b, W1a, b1a, W1b, b1b,
           HW1, Hb1, HW2, Hb2):
    src = edge_index[0].astype(jnp.int32).reshape(NW, NG, GK, CHUNK)
    dst = edge_index[1].astype(jnp.int32).reshape(NW, NG, GK, CHUNK)
    zer = jnp.zeros((ROWS_PER_TILE, D), jnp.float32)

    eps0r = jnp.reshape(eps0, (1, 1))
    eps1r = jnp.reshape(eps1, (1, 1))
    b0ar = b0a.reshape(1, H)
    b0br = b0b.reshape(1, H)
    b1ar = b1a.reshape(1, H)
    b1br = b1b.reshape(1, H)
    HW2p = jnp.pad(HW2, ((0, 0), (0, 0), (0, H - 4)))
    Hb2p = jnp.pad(Hb2, ((0, 0), (0, H - 4)))
    bounds = jnp.array([0.3, 0.3, 0.3, 0.1], jnp.float32)
    bndp = jnp.pad(jnp.broadcast_to(bounds, (NE, 4)), ((0, 0), (0, H - 4)),
                   constant_values=1.0)

    segsum = _make_segsum()
    agg1 = segsum(x, src, dst, zer)
    h1 = _mlp1(eps0r, x, agg1, agg1, W0a, b0ar, W0b, b0br)
    agg2 = segsum(h1, src, dst, zer)
    out = _mlp2(eps1r, h1, agg2, agg2, W1a, b1ar, W1b, b1br,
                HW1, Hb1, HW2p, Hb2p, bndp)
    return out[:, :4]


# speedup vs baseline: 1.1932x; 1.0151x over previous
import functools

import jax
import jax.numpy as jnp
from jax import lax
from jax.experimental import pallas as pl
from jax.experimental.pallas import tpu as pltpu
from jax.experimental.pallas import tpu_sc as plsc

N = 10000
E = 320000
D = 128
H = 128
NE = 8

NC = 2
NS = 16
NW = NC * NS
CHUNK = 125
CPT = E // (NW * CHUNK)
GK = 16
NG = CPT // GK
NSLOT = 2
NPAD = 10240
ROWS_PER_TILE = NPAD // NS
NBLK = 2000
GRID = N // NBLK


def _segsum_body(x_hbm, src_hbm, dst_hbm, zer_hbm, out_hbm,
                 src_v, dst_v, rows_v, acc_sh,
                 semi0, semi1, semr0, semr1, semr2, semr3,
                 sems0, sems1, sems2, sems3):
    c = lax.axis_index("c")
    s = lax.axis_index("s")
    tile = c * NS + s
    semi = (semi0, semi1)
    semr = (semr0, semr1, semr2, semr3)[:NSLOT]
    sems = (sems0, sems1, sems2, sems3)[:NSLOT]

    pltpu.sync_copy(zer_hbm, acc_sh.at[pl.ds(s * ROWS_PER_TILE, ROWS_PER_TILE)])

    pltpu.async_copy(src_hbm.at[tile, 0], src_v.at[0], semi0)
    pltpu.async_copy(dst_hbm.at[tile, 0], dst_v.at[0], semi0)
    plsc.subcore_barrier()
    pltpu.make_async_copy(src_hbm.at[tile, 0], src_v.at[0], semi0).wait()
    pltpu.make_async_copy(dst_hbm.at[tile, 0], dst_v.at[0], semi0).wait()
    pltpu.async_copy(x_hbm.at[src_v.at[0, 0]], rows_v.at[0], semr0)

    def _scatter_wait(rb, gb, k):
        pltpu.make_async_copy(rows_v.at[rb], acc_sh.at[dst_v.at[gb, k]],
                              sems[rb]).wait()

    def _group(g, gb, first):
        for k in range(GK):
            rb = k % NSLOT
            nb = (k + 1) % NSLOT
            if not (first and k < NSLOT - 1):
                if k >= NSLOT - 1:
                    _scatter_wait(nb, gb, k + 1 - NSLOT)
                else:
                    _scatter_wait(nb, 1 - gb, GK + k + 1 - NSLOT)
            if k < GK - 1:
                pltpu.async_copy(x_hbm.at[src_v.at[gb, k + 1]],
                                 rows_v.at[nb], semr[nb])
            else:
                @pl.when(g + 1 < NG)
                def _():
                    pltpu.make_async_copy(src_hbm.at[tile, g + 1],
                                          src_v.at[1 - gb], semi[1 - gb]).wait()
                    pltpu.make_async_copy(dst_hbm.at[tile, g + 1],
                                          dst_v.at[1 - gb], semi[1 - gb]).wait()
                    pltpu.async_copy(x_hbm.at[src_v.at[1 - gb, 0]],
                                     rows_v.at[nb], semr[nb])
            pltpu.make_async_copy(x_hbm.at[src_v.at[gb, k]], rows_v.at[rb],
                                  semr[rb]).wait()
            pltpu.async_copy(rows_v.at[rb], acc_sh.at[dst_v.at[gb, k]],
                             sems[rb], add=True)
            if k == NSLOT - 2:
                @pl.when(g + 1 < NG)
                def _():
                    pltpu.async_copy(src_hbm.at[tile, g + 1], src_v.at[1 - gb],
                                     semi[1 - gb])
                    pltpu.async_copy(dst_hbm.at[tile, g + 1], dst_v.at[1 - gb],
                                     semi[1 - gb])

    _group(0, 0, True)

    @pl.loop(1, NG, step=2)
    def _(g):
        _group(g, 1, False)

        @pl.when(g + 1 < NG)
        def _():
            _group(g + 1, 0, False)

    for k in range(GK - NSLOT + 1, GK):
        _scatter_wait(k % NSLOT, (NG - 1) % 2, k)
    plsc.subcore_barrier()
    pltpu.sync_copy(acc_sh.at[pl.ds(s * ROWS_PER_TILE, ROWS_PER_TILE)],
                    out_hbm.at[c, pl.ds(s * ROWS_PER_TILE, ROWS_PER_TILE)])


@functools.cache
def _make_segsum():
    return pl.kernel(
        _segsum_body,
        out_type=jax.ShapeDtypeStruct((NC, NPAD, D), jnp.float32),
        mesh=plsc.VectorSubcoreMesh(core_axis_name="c", subcore_axis_name="s",
                                    num_cores=NC, num_subcores=NS),
        scratch_types=[
            pltpu.VMEM((2, GK, CHUNK), jnp.int32),
            pltpu.VMEM((2, GK, CHUNK), jnp.int32),
            pltpu.VMEM((NSLOT, CHUNK, D), jnp.float32),
            pltpu.VMEM_SHARED((NPAD, D), jnp.float32),
        ] + [pltpu.SemaphoreType.DMA] * 10,
    )


def _mlp1_body(eps_ref, x_ref, a0_ref, a1_ref, wa_ref, ba_ref, wb_ref, bb_ref, o_ref):
    h = (1.0 + eps_ref[0, 0]) * x_ref[...] + a0_ref[0] + a1_ref[0]
    h = jnp.maximum(jnp.dot(h, wa_ref[...], preferred_element_type=jnp.float32)
                    + ba_ref[...], 0.0)
    h = jnp.maximum(jnp.dot(h, wb_ref[...], preferred_element_type=jnp.float32)
                    + bb_ref[...], 0.0)
    o_ref[...] = h


def _mlp2_body(eps_ref, x_ref, a0_ref, a1_ref, wa_ref, ba_ref, wb_ref, bb_ref,
               hw1_ref, hb1_ref, hw2_ref, hb2_ref, bnd_ref, o_ref, csum_ref):
    i = pl.program_id(0)
    h = (1.0 + eps_ref[0, 0]) * x_ref[...] + a0_ref[0] + a1_ref[0]
    h = jnp.maximum(jnp.dot(h, wa_ref[...], preferred_element_type=jnp.float32)
                    + ba_ref[...], 0.0)
    h = jnp.maximum(jnp.dot(h, wb_ref[...], preferred_element_type=jnp.float32)
                    + bb_ref[...], 0.0)
    cs = jnp.sum(h, axis=0, keepdims=True)

    @pl.when(i == 0)
    def _():
        csum_ref[...] = cs

    @pl.when(i > 0)
    def _():
        csum_ref[...] = csum_ref[...] + cs

    @pl.when(i == GRID - 1)
    def _():
        hp = csum_ref[...] * (1.0 / N)
        rows = []
        for e in range(NE):
            z = jnp.maximum(
                jnp.dot(hp, hw1_ref[e], preferred_element_type=jnp.float32)
                + hb1_ref[e:e + 1, :], 0.0)
            r = jax.nn.sigmoid(
                jnp.dot(z, hw2_ref[e], preferred_element_type=jnp.float32)
                + hb2_ref[e:e + 1, :])
            rows.append(jnp.minimum(r, bnd_ref[e:e + 1, :]))
        o_ref[...] = jnp.concatenate(rows, axis=0)


def _row_blocks():
    return [
        pl.BlockSpec((1, 1), lambda i: (0, 0), memory_space=pltpu.SMEM),
        pl.BlockSpec((NBLK, D), lambda i: (i, 0)),
        pl.BlockSpec((1, NBLK, D), lambda i: (0, i, 0)),
        pl.BlockSpec((1, NBLK, D), lambda i: (1, i, 0)),
        pl.BlockSpec((D, H), lambda i: (0, 0)),
        pl.BlockSpec((1, H), lambda i: (0, 0)),
        pl.BlockSpec((H, H), lambda i: (0, 0)),
        pl.BlockSpec((1, H), lambda i: (0, 0)),
    ]


_mlp1 = pl.pallas_call(
    _mlp1_body,
    grid=(GRID,),
    in_specs=_row_blocks(),
    out_specs=pl.BlockSpec((NBLK, H), lambda i: (i, 0)),
    out_shape=jax.ShapeDtypeStruct((N, H), jnp.float32),
)

_mlp2 = pl.pallas_call(
    _mlp2_body,
    grid=(GRID,),
    in_specs=_row_blocks() + [
        pl.BlockSpec((NE, H, H), lambda i: (0, 0, 0)),
        pl.BlockSpec((NE, H), lambda i: (0, 0)),
        pl.BlockSpec((NE, H, H), lambda i: (0, 0, 0)),
        pl.BlockSpec((NE, H), lambda i: (0, 0)),
        pl.BlockSpec((NE, H), lambda i: (0, 0)),
    ],
    out_specs=pl.BlockSpec((NE, H), lambda i: (0, 0)),
    out_shape=jax.ShapeDtypeStruct((NE, H), jnp.float32),
    scratch_shapes=[pltpu.VMEM((1, H), jnp.float32)],
)


def kernel(x, edge_index, eps0, eps1, W0a, b0a, W0b, b0b, W1a, b1a, W1b, b1b,
           HW1, Hb1, HW2, Hb2):
    src = edge_index[0].astype(jnp.int32).reshape(NW, NG, GK, CHUNK)
    dst = edge_index[1].astype(jnp.int32).reshape(NW, NG, GK, CHUNK)
    zer = jnp.zeros((ROWS_PER_TILE, D), jnp.float32)

    eps0r = jnp.reshape(eps0, (1, 1))
    eps1r = jnp.reshape(eps1, (1, 1))
    b0ar = b0a.reshape(1, H)
    b0br = b0b.reshape(1, H)
    b1ar = b1a.reshape(1, H)
    b1br = b1b.reshape(1, H)
    HW2p = jnp.pad(HW2, ((0, 0), (0, 0), (0, H - 4)))
    Hb2p = jnp.pad(Hb2, ((0, 0), (0, H - 4)))
    bounds = jnp.array([0.3, 0.3, 0.3, 0.1], jnp.float32)
    bndp = jnp.pad(jnp.broadcast_to(bounds, (NE, 4)), ((0, 0), (0, H - 4)),
                   constant_values=1.0)

    segsum = _make_segsum()
    agg1 = segsum(x, src, dst, zer)
    h1 = _mlp1(eps0r, x, agg1, agg1, W0a, b0ar, W0b, b0br)
    agg2 = segsum(h1, src, dst, zer)
    out = _mlp2(eps1r, h1, agg2, agg2, W1a, b1ar, W1b, b1br,
                HW1, Hb1, HW2p, Hb2p, bndp)
    return out[:, :4]


# speedup vs baseline: 1.2407x; 1.0398x over previous
import functools

import jax
import jax.numpy as jnp
from jax import lax
from jax.experimental import pallas as pl
from jax.experimental.pallas import tpu as pltpu
from jax.experimental.pallas import tpu_sc as plsc

N = 10000
E = 320000
D = 128
H = 128
NE = 8

NC = 2
NS = 16
NW = NC * NS
CHUNK = 125
CPT = E // (NW * CHUNK)
GK = 16
NG = CPT // GK
NSLOT = 2
NPAD = 10240
ROWS_PER_TILE = NPAD // NS
NBLK = 2000
GRID = N // NBLK


def _segsum_body(x_hbm, src_hbm, dst_hbm, out_hbm,
                 src_v, dst_v, rows_v, acc_sh,
                 semi0, semi1, semr0, semr1, semr2, semr3,
                 sems0, sems1, sems2, sems3):
    c = lax.axis_index("c")
    s = lax.axis_index("s")
    tile = c * NS + s
    semi = (semi0, semi1)
    semr = (semr0, semr1, semr2, semr3)[:NSLOT]
    sems = (sems0, sems1, sems2, sems3)[:NSLOT]

    zv = jnp.zeros((16,), jnp.float32)

    @pl.loop(0, 80)
    def _(i):
        for j in range(8):
            rows_v[0, i, pl.ds(j * 16, 16)] = zv

    for z in range(ROWS_PER_TILE // 80):
        pltpu.sync_copy(rows_v.at[0, pl.ds(0, 80)],
                        acc_sh.at[pl.ds(s * ROWS_PER_TILE + z * 80, 80)])

    pltpu.async_copy(src_hbm.at[tile, 0], src_v.at[0], semi0)
    pltpu.async_copy(dst_hbm.at[tile, 0], dst_v.at[0], semi0)
    plsc.subcore_barrier()
    pltpu.make_async_copy(src_hbm.at[tile, 0], src_v.at[0], semi0).wait()
    pltpu.make_async_copy(dst_hbm.at[tile, 0], dst_v.at[0], semi0).wait()
    pltpu.async_copy(x_hbm.at[src_v.at[0, 0]], rows_v.at[0], semr0)

    def _scatter_wait(rb, gb, k):
        pltpu.make_async_copy(rows_v.at[rb], acc_sh.at[dst_v.at[gb, k]],
                              sems[rb]).wait()

    def _group(g, gb, first):
        for k in range(GK):
            rb = k % NSLOT
            nb = (k + 1) % NSLOT
            if not (first and k < NSLOT - 1):
                if k >= NSLOT - 1:
                    _scatter_wait(nb, gb, k + 1 - NSLOT)
                else:
                    _scatter_wait(nb, 1 - gb, GK + k + 1 - NSLOT)
            if k < GK - 1:
                pltpu.async_copy(x_hbm.at[src_v.at[gb, k + 1]],
                                 rows_v.at[nb], semr[nb])
            else:
                @pl.when(g + 1 < NG)
                def _():
                    pltpu.make_async_copy(src_hbm.at[tile, g + 1],
                                          src_v.at[1 - gb], semi[1 - gb]).wait()
                    pltpu.make_async_copy(dst_hbm.at[tile, g + 1],
                                          dst_v.at[1 - gb], semi[1 - gb]).wait()
                    pltpu.async_copy(x_hbm.at[src_v.at[1 - gb, 0]],
                                     rows_v.at[nb], semr[nb])
            pltpu.make_async_copy(x_hbm.at[src_v.at[gb, k]], rows_v.at[rb],
                                  semr[rb]).wait()
            pltpu.async_copy(rows_v.at[rb], acc_sh.at[dst_v.at[gb, k]],
                             sems[rb], add=True)
            if k == NSLOT - 2:
                @pl.when(g + 1 < NG)
                def _():
                    pltpu.async_copy(src_hbm.at[tile, g + 1], src_v.at[1 - gb],
                                     semi[1 - gb])
                    pltpu.async_copy(dst_hbm.at[tile, g + 1], dst_v.at[1 - gb],
                                     semi[1 - gb])

    _group(0, 0, True)

    @pl.loop(1, NG, step=2)
    def _(g):
        _group(g, 1, False)

        @pl.when(g + 1 < NG)
        def _():
            _group(g + 1, 0, False)

    for k in range(GK - NSLOT + 1, GK):
        _scatter_wait(k % NSLOT, (NG - 1) % 2, k)
    plsc.subcore_barrier()
    pltpu.sync_copy(acc_sh.at[pl.ds(s * ROWS_PER_TILE, ROWS_PER_TILE)],
                    out_hbm.at[c, pl.ds(s * ROWS_PER_TILE, ROWS_PER_TILE)])


@functools.cache
def _make_segsum():
    return pl.kernel(
        _segsum_body,
        out_type=jax.ShapeDtypeStruct((NC, NPAD, D), jnp.float32),
        mesh=plsc.VectorSubcoreMesh(core_axis_name="c", subcore_axis_name="s",
                                    num_cores=NC, num_subcores=NS),
        scratch_types=[
            pltpu.VMEM((2, GK, CHUNK), jnp.int32),
            pltpu.VMEM((2, GK, CHUNK), jnp.int32),
            pltpu.VMEM((NSLOT, CHUNK, D), jnp.float32),
            pltpu.VMEM_SHARED((NPAD, D), jnp.float32),
        ] + [pltpu.SemaphoreType.DMA] * 10,
    )


def _mlp1_body(eps_ref, x_ref, a0_ref, a1_ref, wa_ref, ba_ref, wb_ref, bb_ref, o_ref):
    h = (1.0 + eps_ref[0, 0]) * x_ref[...] + a0_ref[0] + a1_ref[0]
    h = jnp.maximum(jnp.dot(h, wa_ref[...], preferred_element_type=jnp.float32)
                    + ba_ref[...], 0.0)
    h = jnp.maximum(jnp.dot(h, wb_ref[...], preferred_element_type=jnp.float32)
                    + bb_ref[...], 0.0)
    o_ref[...] = h


def _mlp2_body(eps_ref, x_ref, a0_ref, a1_ref, wa_ref, ba_ref, wb_ref, bb_ref,
               hw1_ref, hb1_ref, hw2_ref, hb2_ref, bnd_ref, o_ref, csum_ref):
    i = pl.program_id(0)
    h = (1.0 + eps_ref[0, 0]) * x_ref[...] + a0_ref[0] + a1_ref[0]
    h = jnp.maximum(jnp.dot(h, wa_ref[...], preferred_element_type=jnp.float32)
                    + ba_ref[...], 0.0)
    h = jnp.maximum(jnp.dot(h, wb_ref[...], preferred_element_type=jnp.float32)
                    + bb_ref[...], 0.0)
    cs = jnp.sum(h, axis=0, keepdims=True)

    @pl.when(i == 0)
    def _():
        csum_ref[...] = cs

    @pl.when(i > 0)
    def _():
        csum_ref[...] = csum_ref[...] + cs

    @pl.when(i == GRID - 1)
    def _():
        hp = csum_ref[...] * (1.0 / N)
        rows = []
        for e in range(NE):
            z = jnp.maximum(
                jnp.dot(hp, hw1_ref[e], preferred_element_type=jnp.float32)
                + hb1_ref[e:e + 1, :], 0.0)
            r = jax.nn.sigmoid(
                jnp.dot(z, hw2_ref[e], preferred_element_type=jnp.float32)
                + hb2_ref[e:e + 1, :])
            rows.append(jnp.minimum(r, bnd_ref[e:e + 1, :]))
        o_ref[...] = jnp.concatenate(rows, axis=0)


def _row_blocks():
    return [
        pl.BlockSpec((1, 1), lambda i: (0, 0), memory_space=pltpu.SMEM),
        pl.BlockSpec((NBLK, D), lambda i: (i, 0)),
        pl.BlockSpec((1, NBLK, D), lambda i: (0, i, 0)),
        pl.BlockSpec((1, NBLK, D), lambda i: (1, i, 0)),
        pl.BlockSpec((D, H), lambda i: (0, 0)),
        pl.BlockSpec((1, H), lambda i: (0, 0)),
        pl.BlockSpec((H, H), lambda i: (0, 0)),
        pl.BlockSpec((1, H), lambda i: (0, 0)),
    ]


_mlp1 = pl.pallas_call(
    _mlp1_body,
    grid=(GRID,),
    in_specs=_row_blocks(),
    out_specs=pl.BlockSpec((NBLK, H), lambda i: (i, 0)),
    out_shape=jax.ShapeDtypeStruct((N, H), jnp.float32),
)

_mlp2 = pl.pallas_call(
    _mlp2_body,
    grid=(GRID,),
    in_specs=_row_blocks() + [
        pl.BlockSpec((NE, H, H), lambda i: (0, 0, 0)),
        pl.BlockSpec((NE, H), lambda i: (0, 0)),
        pl.BlockSpec((NE, H, H), lambda i: (0, 0, 0)),
        pl.BlockSpec((NE, H), lambda i: (0, 0)),
        pl.BlockSpec((NE, H), lambda i: (0, 0)),
    ],
    out_specs=pl.BlockSpec((NE, H), lambda i: (0, 0)),
    out_shape=jax.ShapeDtypeStruct((NE, H), jnp.float32),
    scratch_shapes=[pltpu.VMEM((1, H), jnp.float32)],
)


def kernel(x, edge_index, eps0, eps1, W0a, b0a, W0b, b0b, W1a, b1a, W1b, b1b,
           HW1, Hb1, HW2, Hb2):
    src = edge_index[0].astype(jnp.int32).reshape(NW, NG, GK, CHUNK)
    dst = edge_index[1].astype(jnp.int32).reshape(NW, NG, GK, CHUNK)

    eps0r = jnp.reshape(eps0, (1, 1))
    eps1r = jnp.reshape(eps1, (1, 1))
    b0ar = b0a.reshape(1, H)
    b0br = b0b.reshape(1, H)
    b1ar = b1a.reshape(1, H)
    b1br = b1b.reshape(1, H)
    HW2p = jnp.pad(HW2, ((0, 0), (0, 0), (0, H - 4)))
    Hb2p = jnp.pad(Hb2, ((0, 0), (0, H - 4)))
    bounds = jnp.array([0.3, 0.3, 0.3, 0.1], jnp.float32)
    bndp = jnp.pad(jnp.broadcast_to(bounds, (NE, 4)), ((0, 0), (0, H - 4)),
                   constant_values=1.0)

    segsum = _make_segsum()
    agg1 = segsum(x, src, dst)
    h1 = _mlp1(eps0r, x, agg1, agg1, W0a, b0ar, W0b, b0br)
    agg2 = segsum(h1, src, dst)
    out = _mlp2(eps1r, h1, agg2, agg2, W1a, b1ar, W1b, b1br,
                HW1, Hb1, HW2p, Hb2p, bndp)
    return out[:, :4]
